# Initial kernel scaffold; baseline (speedup 1.0000x reference)
import jax
import jax.numpy as jnp
from jax.experimental import pallas as pl


def kernel(x, edge_index, W1, att_src1, att_dst1, b1, W2, att_src2, att_dst2, b2):
    raise NotImplementedError("write your pallas kernel here")


# speedup vs baseline: 3.9210x; 3.9210x over previous
import functools

import jax
import jax.numpy as jnp
from jax import lax
from jax.experimental import pallas as pl
from jax.experimental.pallas import tpu as pltpu
from jax.experimental.pallas import tpu_sc as plsc

NN = 10000
EE = 320000
EREAL = EE + NN
EA = 331776
NP_ = 10240
NC, NS, L = 2, 16, 16
NW = NC * NS
CH32 = EA // NW
CH16 = EA // NS


def _mesh():
    return plsc.VectorSubcoreMesh(
        core_axis_name="c", subcore_axis_name="s", num_cores=NC, num_subcores=NS
    )


def _leaky(v):
    return jnp.maximum(v, 0.2 * v)


def _make_k1(H):
    FLAT = H * NP_
    ZCH = FLAT // NS
    NBLK = CH32 // 128

    @functools.partial(
        pl.kernel,
        out_type=jax.ShapeDtypeStruct((NC, FLAT), jnp.float32),
        mesh=_mesh(),
        compiler_params=pltpu.CompilerParams(needs_layout_passes=False),
        scratch_types=[
            pltpu.VMEM((CH32,), jnp.int32),
            pltpu.VMEM((CH32,), jnp.int32),
            pltpu.VMEM((NP_,), jnp.float32),
            pltpu.VMEM((NP_,), jnp.float32),
            pltpu.VMEM((16,), jnp.float32),
            pltpu.VMEM((16,), jnp.float32),
            pltpu.VMEM((128,), jnp.int32),
            pltpu.VMEM((128,), jnp.float32),
            pltpu.VMEM((ZCH,), jnp.float32),
            pltpu.VMEM_SHARED((FLAT,), jnp.float32),
            pltpu.SemaphoreType.DMA,
        ],
    )
    def k1(src_h, dst_h, asrcT_h, adstT_h, maxs_h, maxd_h, out_h,
           src_v, dst_v, tabs, tabd, maxs_v, maxd_v, idxb, valb, zb,
           den_sh, sem):
        c = lax.axis_index("c")
        s = lax.axis_index("s")
        wid = c * NS + s
        base = pl.multiple_of(wid * CH32, 128)

        def zbody(i, _):
            zb[pl.ds(i * 16, 16)] = jnp.zeros((16,), jnp.float32)
            return 0
        lax.fori_loop(0, ZCH // 16, zbody, 0)
        zoff = pl.multiple_of(s * ZCH, 128)
        pltpu.sync_copy(zb, den_sh.at[pl.ds(zoff, ZCH)])

        pltpu.sync_copy(src_h.at[pl.ds(base, CH32)], src_v)
        pltpu.sync_copy(dst_h.at[pl.ds(base, CH32)], dst_v)
        pltpu.sync_copy(maxs_h, maxs_v)
        pltpu.sync_copy(maxd_h, maxd_v)
        plsc.subcore_barrier()

        msv = maxs_v[pl.ds(0, 16)]
        mdv = maxd_v[pl.ds(0, 16)]
        for h in range(H):
            pltpu.sync_copy(asrcT_h.at[h], tabs)
            pltpu.sync_copy(adstT_h.at[h], tabd)
            M = _leaky(msv[h] + mdv[h])

            def blk(b, _):
                for j in range(8):
                    off = b * 128 + j * 16
                    s16 = src_v[pl.ds(off, 16)]
                    d16 = dst_v[pl.ds(off, 16)]
                    av = plsc.load_gather(tabs, [s16])
                    bv = plsc.load_gather(tabd, [d16])
                    ex = jnp.exp(_leaky(av + bv) - M)
                    gpos = base + off + lax.iota(jnp.int32, 16)
                    ex = jnp.where(gpos < EREAL, ex, 0.0)
                    idxb[pl.ds(j * 16, 16)] = d16 + h * NP_
                    valb[pl.ds(j * 16, 16)] = ex
                pltpu.async_copy(valb, den_sh.at[idxb], sem, add=True).wait()
                return 0
            lax.fori_loop(0, NBLK, blk, 0)

        plsc.subcore_barrier()
        pltpu.sync_copy(den_sh.at[pl.ds(zoff, ZCH)],
                        out_h.at[c].at[pl.ds(zoff, ZCH)])

    return k1


def _make_k2(H):
    SUB = 3456
    NSUB = CH32 // SUB

    @functools.partial(
        pl.kernel,
        out_type=jax.ShapeDtypeStruct((EA // 8, 128), jnp.float32),
        mesh=_mesh(),
        compiler_params=pltpu.CompilerParams(needs_layout_passes=False),
        scratch_types=[
            pltpu.VMEM((SUB,), jnp.int32),
            pltpu.VMEM((SUB,), jnp.int32),
            pltpu.VMEM((NP_,), jnp.float32),
            pltpu.VMEM((NP_,), jnp.float32),
            pltpu.VMEM((NP_,), jnp.float32),
            pltpu.VMEM((NP_,), jnp.float32),
            pltpu.VMEM((16,), jnp.float32),
            pltpu.VMEM((16,), jnp.float32),
            pltpu.VMEM((SUB // 8, 128), jnp.float32),
        ],
    )
    def k2(src_h, dst_h, asrcT_h, adstT_h, maxs_h, maxd_h, den_h, alpha_h,
           srcb, dstb, tabs, tabd, tden, tden2, maxs_v, maxd_v, stg):
        c = lax.axis_index("c")
        s = lax.axis_index("s")
        wid = c * NS + s
        base = pl.multiple_of(wid * CH32, 128)
        pltpu.sync_copy(maxs_h, maxs_v)
        pltpu.sync_copy(maxd_h, maxd_v)
        msv = maxs_v[pl.ds(0, 16)]
        mdv = maxd_v[pl.ds(0, 16)]

        for sub in range(NSUB):
            off0 = pl.multiple_of(base + sub * SUB, 128)
            pltpu.sync_copy(src_h.at[pl.ds(off0, SUB)], srcb)
            pltpu.sync_copy(dst_h.at[pl.ds(off0, SUB)], dstb)
            for h in range(H):
                pltpu.sync_copy(asrcT_h.at[h], tabs)
                pltpu.sync_copy(adstT_h.at[h], tabd)
                pltpu.sync_copy(den_h.at[0].at[pl.ds(h * NP_, NP_)], tden)
                pltpu.sync_copy(den_h.at[1].at[pl.ds(h * NP_, NP_)], tden2)

                def dadd(i, _):
                    sl = pl.ds(i * 16, 16)
                    tden[sl] = tden[sl] + tden2[sl]
                    return 0
                lax.fori_loop(0, NP_ // 16, dadd, 0)

                M = _leaky(msv[h] + mdv[h])

                def body(j, _):
                    o = j * 16
                    s16 = srcb[pl.ds(o, 16)]
                    d16 = dstb[pl.ds(o, 16)]
                    av = plsc.load_gather(tabs, [s16])
                    bv = plsc.load_gather(tabd, [d16])
                    ex = jnp.exp(_leaky(av + bv) - M)
                    den = plsc.load_gather(tden, [d16])
                    al = ex / (den + 1e-16)
                    gpos = off0 + o + lax.iota(jnp.int32, 16)
                    al = jnp.where(gpos < EREAL, al, 0.0)
                    le16 = o + lax.iota(jnp.int32, 16)
                    plsc.store_scatter(
                        stg, [le16 >> 3, ((le16 & 7) << 4) + h], al)
                    return 0
                lax.fori_loop(0, SUB // 16, body, 0)
            ro = pl.multiple_of(off0 // 8, 8)
            pltpu.sync_copy(stg, alpha_h.at[pl.ds(ro, SUB // 8)])

    return k2


def _make_k3(H, D, NPASS, PART_T, OB, RB):
    PF = max(1, 128 // D)
    PSH = PF.bit_length() - 1
    NOB = CH16 * NS // OB
    DH = D // H

    @functools.partial(
        pl.kernel,
        out_type=jax.ShapeDtypeStruct((NP_ * D,), jnp.float32),
        mesh=_mesh(),
        compiler_params=pltpu.CompilerParams(needs_layout_passes=False),
        scratch_types=[
            pltpu.VMEM((OB,), jnp.int32),
            pltpu.VMEM((OB,), jnp.int32),
            pltpu.VMEM((OB + 128,), jnp.int32),
            pltpu.VMEM((OB + 128,), jnp.int32),
            pltpu.VMEM((OB + 128,), jnp.int32),
            pltpu.VMEM((RB,), jnp.int32),
            pltpu.VMEM((RB,), jnp.int32),
            pltpu.VMEM((RB, PF * D), jnp.float32),
            pltpu.VMEM((RB, 128), jnp.float32),
            pltpu.VMEM((PART_T * D,), jnp.float32),
            pltpu.SemaphoreType.DMA,
            pltpu.SemaphoreType.DMA,
        ],
    )
    def k3(src_h, dst_h, alpha_h, feat_h, out_h,
           srcb, dstb, sel_s, sel_d, sel_e, gix, aix, rows, arows, acc,
           sem, sem2):
        c = lax.axis_index("c")
        s = lax.axis_index("s")
        z16i = jnp.zeros((16,), jnp.int32)
        z16f = jnp.zeros((16,), jnp.float32)
        pad16e = jnp.full((16,), EREAL, jnp.int32)

        for p in range(NPASS):
            lo = ((c * NPASS + p) * NS + s) * PART_T

            def zacc(i, _):
                acc[pl.ds(i * 16, 16)] = z16f
                return 0
            lax.fori_loop(0, PART_T * D // 16, zacc, 0)

            def ob_body(ob, _):
                off0 = pl.multiple_of(ob * OB, 128)
                pltpu.sync_copy(src_h.at[pl.ds(off0, OB)], srcb)
                pltpu.sync_copy(dst_h.at[pl.ds(off0, OB)], dstb)

                def compact(j, cnt):
                    o = j * 16
                    d16 = dstb[pl.ds(o, 16)]
                    m = (d16 >= lo) & (d16 < lo + PART_T)
                    s16 = srcb[pl.ds(o, 16)]
                    e16 = off0 + o + lax.iota(jnp.int32, 16)
                    plsc.store_compressed(sel_s.at[pl.ds(cnt, 16)], s16,
                                          mask=m)
                    plsc.store_compressed(sel_d.at[pl.ds(cnt, 16)], d16 - lo,
                                          mask=m)
                    plsc.store_compressed(sel_e.at[pl.ds(cnt, 16)], e16,
                                          mask=m)
                    pc = plsc.all_reduce_population_count(m)
                    return cnt + jnp.max(pc)
                cnt = lax.fori_loop(0, OB // 16, compact, 0)

                for k in range(RB // 16):
                    sel_s[pl.ds(cnt + k * 16, 16)] = z16i
                    sel_d[pl.ds(cnt + k * 16, 16)] = z16i
                    sel_e[pl.ds(cnt + k * 16, 16)] = pad16e

                nb = (cnt + RB - 1) // RB

                def bblk(b, _):
                    for k in range(RB // 16):
                        gix[pl.ds(k * 16, 16)] = (
                            sel_s[pl.ds(b * RB + k * 16, 16)] >> PSH)
                        aix[pl.ds(k * 16, 16)] = (
                            sel_e[pl.ds(b * RB + k * 16, 16)] >> 3)
                    cp1 = pltpu.async_copy(feat_h.at[gix], rows, sem)
                    cp2 = pltpu.async_copy(alpha_h.at[aix], arows, sem2)
                    cp1.wait()
                    cp2.wait()

                    def gbody(g, _):
                        dl16 = sel_d[pl.ds(b * RB + g * 16, 16)]
                        eb16 = sel_e[pl.ds(b * RB + g * 16, 16)]
                        sv16 = sel_s[pl.ds(b * RB + g * 16, 16)]
                        for k in range(16):
                            r = g * 16 + k
                            av = arows[r, pl.ds((eb16[k] & 7) << 4, 16)]
                            dbase = dl16[k] * D
                            cbase = (sv16[k] & (PF - 1)) * D if PF > 1 else 0
                            for h in range(H):
                                a = av[h]
                                for q in range(DH // 16):
                                    col = h * DH + q * 16
                                    v = rows[r, pl.ds(cbase + col, 16)] * a
                                    plsc.addupdate(
                                        acc.at[pl.ds(dbase + col, 16)], v)
                        return 0
                    lax.fori_loop(0, RB // 16, gbody, 0)
                    return 0
                lax.fori_loop(0, nb, bblk, 0)
                return 0
            lax.fori_loop(0, NOB, ob_body, 0)

            ooff = pl.multiple_of(lo * D, 128)
            pltpu.sync_copy(acc, out_h.at[pl.ds(ooff, PART_T * D)])

    return k3


def _tc_embed(x, W, amS, amD, relu_bias=None, R=1000):
    N_, DIN = x.shape
    HD = W.shape[1]
    grid = (N_ // R,)

    def body(*refs):
        if relu_bias is None:
            x_r, w_r, ams_r, amd_r, h_r, as_r, ad_r, ms_r, md_r = refs
            xv = x_r[...]
        else:
            x_r, b_r, w_r, ams_r, amd_r, h_r, as_r, ad_r, ms_r, md_r = refs
            xv = jnp.maximum(x_r[...] + b_r[...], 0.0)
        i = pl.program_id(0)
        h = jnp.dot(xv, w_r[...], preferred_element_type=jnp.float32)
        h_r[...] = h
        a_s = jnp.dot(h, ams_r[...], preferred_element_type=jnp.float32)
        a_d = jnp.dot(h, amd_r[...], preferred_element_type=jnp.float32)
        as_r[...] = a_s
        ad_r[...] = a_d
        bs = jnp.max(a_s, axis=0, keepdims=True)
        bd = jnp.max(a_d, axis=0, keepdims=True)

        @pl.when(i == 0)
        def _():
            ms_r[...] = bs
            md_r[...] = bd

        @pl.when(i > 0)
        def _():
            ms_r[...] = jnp.maximum(ms_r[...], bs)
            md_r[...] = jnp.maximum(md_r[...], bd)

    in_specs = [pl.BlockSpec((R, DIN), lambda i: (i, 0))]
    args = [x]
    if relu_bias is not None:
        in_specs.append(pl.BlockSpec((1, DIN), lambda i: (0, 0)))
        args.append(relu_bias.reshape(1, DIN))
    in_specs += [
        pl.BlockSpec((DIN, HD), lambda i: (0, 0)),
        pl.BlockSpec((HD, 8), lambda i: (0, 0)),
        pl.BlockSpec((HD, 8), lambda i: (0, 0)),
    ]
    args += [W, amS, amD]
    return pl.pallas_call(
        body,
        grid=grid,
        in_specs=in_specs,
        out_specs=[
            pl.BlockSpec((R, HD), lambda i: (i, 0)),
            pl.BlockSpec((R, 8), lambda i: (i, 0)),
            pl.BlockSpec((R, 8), lambda i: (i, 0)),
            pl.BlockSpec((1, 8), lambda i: (0, 0)),
            pl.BlockSpec((1, 8), lambda i: (0, 0)),
        ],
        out_shape=[
            jax.ShapeDtypeStruct((N_, HD), jnp.float32),
            jax.ShapeDtypeStruct((N_, 8), jnp.float32),
            jax.ShapeDtypeStruct((N_, 8), jnp.float32),
            jax.ShapeDtypeStruct((1, 8), jnp.float32),
            jax.ShapeDtypeStruct((1, 8), jnp.float32),
        ],
    )(*args)


def _tc_bias(agg, b, R=1000):
    N_, D_ = agg.shape

    def body(a_r, b_r, o_r):
        o_r[...] = a_r[...] + b_r[...]

    return pl.pallas_call(
        body,
        grid=(N_ // R,),
        in_specs=[
            pl.BlockSpec((R, D_), lambda i: (i, 0)),
            pl.BlockSpec((1, D_), lambda i: (0, 0)),
        ],
        out_specs=pl.BlockSpec((R, D_), lambda i: (i, 0)),
        out_shape=jax.ShapeDtypeStruct((N_, D_), jnp.float32),
    )(agg, b.reshape(1, D_))


def _att_mat(att, H):
    a = att.reshape(H, -1)
    C = a.shape[1]
    m = a[:, :, None] * jnp.eye(H, 8, dtype=jnp.float32)[:, None, :]
    return m.reshape(H * C, 8)


def _pad16(m):
    return jnp.concatenate([m.reshape(8), jnp.zeros((8,), jnp.float32)])


def kernel(x, edge_index, W1, att_src1, att_dst1, b1, W2, att_src2, att_dst2,
           b2):
    loop = jnp.arange(NN, dtype=jnp.int32)
    zpad = jnp.zeros((EA - EREAL,), jnp.int32)
    src = jnp.concatenate([edge_index[0], loop, zpad])
    dst = jnp.concatenate([edge_index[1], loop, zpad])

    amS1 = _att_mat(att_src1, 8)
    amD1 = _att_mat(att_dst1, 8)
    h1, a_s1, a_d1, ms1, md1 = _tc_embed(x, W1, amS1, amD1)
    asrcT1 = jnp.pad(a_s1.T, ((0, 0), (0, NP_ - NN)))
    adstT1 = jnp.pad(a_d1.T, ((0, 0), (0, NP_ - NN)))
    maxs1 = _pad16(ms1)
    maxd1 = _pad16(md1)

    den1 = _make_k1(8)(src, dst, asrcT1, adstT1, maxs1, maxd1)
    alpha1 = _make_k2(8)(src, dst, asrcT1, adstT1, maxs1, maxd1, den1)
    agg1 = _make_k3(8, 512, 2, 160, 1152, 32)(src, dst, alpha1, h1)
    agg1 = agg1.reshape(NP_, 512)

    amS2 = jnp.concatenate(
        [att_src2.reshape(64, 1), jnp.zeros((64, 7), jnp.float32)], axis=1)
    amD2 = jnp.concatenate(
        [att_dst2.reshape(64, 1), jnp.zeros((64, 7), jnp.float32)], axis=1)
    h2p, a_s2, a_d2, ms2, md2 = _tc_embed(agg1, W2, amS2, amD2,
                                          relu_bias=b1, R=1024)
    h2 = h2p[:NN]
    asrcT2 = jnp.pad(a_s2[:NN, :1].T, ((0, 0), (0, NP_ - NN)))
    adstT2 = jnp.pad(a_d2[:NN, :1].T, ((0, 0), (0, NP_ - NN)))
    maxs2 = _pad16(ms2)
    maxd2 = _pad16(md2)

    den2 = _make_k1(1)(src, dst, asrcT2, adstT2, maxs2, maxd2)
    alpha2 = _make_k2(1)(src, dst, asrcT2, adstT2, maxs2, maxd2, den2)
    h2pk = h2.reshape(NN // 2, 128)
    agg2 = _make_k3(1, 64, 1, 320, 3456, 64)(src, dst, alpha2, h2pk)
    agg2 = agg2.reshape(NP_, 64)

    return _tc_bias(agg2[:NN], b2)
